# baseline (speedup 1.0000x reference)
import jax
import jax.numpy as jnp
from jax import lax
from jax.experimental import pallas as pl
from jax.experimental.pallas import tpu as pltpu
from jax.experimental.pallas import tpu_sc as plsc

N_NODES = 50000
N_EDGES = 1600000
N_HEAD = 4
D_HEAD = 8
DQK = N_HEAD * D_HEAD
ROWS = N_EDGES // 4
OROWS = N_EDGES * N_HEAD // 128

N_SLOTS = 8
CROWS = 1600
CHUNK = CROWS * 4
N_CHUNKS = N_EDGES // CHUNK
MAX_J = (N_CHUNKS + N_SLOTS - 1) // N_SLOTS
NPAD = 51200
LANES = 16


def _score_body(q_ref, k_ref, wq_ref, wk_ref, p_ref, o_ref):
    u = q_ref[...] * wq_ref[...] + k_ref[...] * wk_ref[...]
    hi = u.astype(jnp.bfloat16)
    lo = (u - hi.astype(jnp.float32)).astype(jnp.bfloat16)
    p = p_ref[...]
    s = lax.dot_general(p, hi, (((1,), (0,)), ((), ())),
                        preferred_element_type=jnp.float32)
    s = s + lax.dot_general(p, lo, (((1,), (0,)), ((), ())),
                            preferred_element_type=jnp.float32)
    o_ref[...] = jnp.exp(jnp.maximum(s, 0.0))


def _stage1(qT, kT, wqc, wkc, psum, be=12800):
    grid = N_EDGES // be
    return pl.pallas_call(
        _score_body,
        grid=(grid,),
        in_specs=[
            pl.BlockSpec((DQK, be), lambda i: (0, i)),
            pl.BlockSpec((DQK, be), lambda i: (0, i)),
            pl.BlockSpec((DQK, 1), lambda i: (0, 0)),
            pl.BlockSpec((DQK, 1), lambda i: (0, 0)),
            pl.BlockSpec((N_HEAD, DQK), lambda i: (0, 0)),
        ],
        out_specs=pl.BlockSpec((N_HEAD, be), lambda i: (0, i)),
        out_shape=jax.ShapeDtypeStruct((N_HEAD, N_EDGES), jnp.float32),
    )(qT, kT, wqc, wkc, psum)


def _scatter_body(ex4_hbm, idx_hbm, part_hbm, table, idx_buf, val4_buf):
    wid = lax.axis_index("s") * 2 + lax.axis_index("c")
    head = wid // N_SLOTS
    slot = wid % N_SLOTS

    def zero_step(i, _):
        table[i >> 3, pl.ds((i & 7) * LANES, LANES)] = jnp.zeros((LANES,), jnp.float32)
        return 0
    lax.fori_loop(0, NPAD // LANES, zero_step, 0)

    def chunk_step(j, _):
        c = slot + j * N_SLOTS

        @pl.when(c < N_CHUNKS)
        def _():
            pltpu.sync_copy(idx_hbm.at[pl.ds(c * CHUNK, CHUNK)], idx_buf)
            pltpu.sync_copy(ex4_hbm.at[:, pl.ds(c * CHUNK, CHUNK)], val4_buf)

            def scat_step(t, _):
                iv = idx_buf[pl.ds(t * LANES, LANES)]
                xv = val4_buf[head, pl.ds(t * LANES, LANES)]
                plsc.addupdate_scatter(table, [iv >> 7, iv & 127], xv)
                return 0
            lax.fori_loop(0, CHUNK // LANES, scat_step, 0, unroll=4)
        return 0
    lax.fori_loop(0, MAX_J, chunk_step, 0)

    pltpu.sync_copy(table, part_hbm.at[pl.ds(wid * (NPAD // 128), NPAD // 128), :])


def _stage2(ex4, idx):
    mesh = plsc.VectorSubcoreMesh(core_axis_name="c", subcore_axis_name="s")
    f = pl.kernel(
        _scatter_body,
        out_type=jax.ShapeDtypeStruct((32 * NPAD // 128, 128), jnp.float32),
        mesh=mesh,
        scratch_types=[
            pltpu.VMEM((NPAD // 128, 128), jnp.float32),
            pltpu.VMEM((CHUNK,), jnp.int32),
            pltpu.VMEM((N_HEAD, CHUNK), jnp.float32),
        ],
        compiler_params=pltpu.CompilerParams(needs_layout_passes=False),
    )
    return f(ex4, idx)


def _inv_body(p_ref, o_ref):
    p = p_ref[...].reshape(N_HEAD, N_SLOTS, NPAD // 128, 128)
    d = jnp.sum(p, axis=1)
    o_ref[...] = (1.0 / (d + 1e-16)).reshape(N_HEAD * NPAD // 128, 128)


def _stage3(partials):
    return pl.pallas_call(
        _inv_body,
        out_shape=jax.ShapeDtypeStruct((N_HEAD * NPAD // 128, 128), jnp.float32),
    )(partials)


def _gather_body(inv_hbm, idx_hbm, ex4_hbm, out_hbm, table, idx_buf, val4_buf, g_buf):
    wid = lax.axis_index("s") * 2 + lax.axis_index("c")
    head = wid // N_SLOTS
    slot = wid % N_SLOTS

    pltpu.sync_copy(inv_hbm.at[pl.ds(head * (NPAD // 128), NPAD // 128), :], table)
    def chunk_step(j, _):
        c = slot + j * N_SLOTS

        @pl.when(c < N_CHUNKS)
        def _():
            pltpu.sync_copy(idx_hbm.at[pl.ds(c * CHUNK, CHUNK)], idx_buf)
            pltpu.sync_copy(ex4_hbm.at[:, pl.ds(c * CHUNK, CHUNK)], val4_buf)

            def gat_step(t, _):
                iv = idx_buf[pl.ds(t * LANES, LANES)]
                gv = plsc.load_gather(table, [iv >> 7, iv & 127])
                xv = val4_buf[head, pl.ds(t * LANES, LANES)]
                g_buf[pl.ds(t * LANES, LANES)] = gv * xv
                return 0
            lax.fori_loop(0, CHUNK // LANES, gat_step, 0, unroll=4)
            pltpu.sync_copy(g_buf, out_hbm.at[pl.ds(head * N_EDGES + c * CHUNK, CHUNK)])
        return 0
    lax.fori_loop(0, MAX_J, chunk_step, 0)


def _stage4(inv, idx, ex4):
    mesh = plsc.VectorSubcoreMesh(core_axis_name="c", subcore_axis_name="s")
    f = pl.kernel(
        _gather_body,
        out_type=jax.ShapeDtypeStruct((N_HEAD * N_EDGES,), jnp.float32),
        mesh=mesh,
        scratch_types=[
            pltpu.VMEM((NPAD // 128, 128), jnp.float32),
            pltpu.VMEM((CHUNK,), jnp.int32),
            pltpu.VMEM((N_HEAD, CHUNK), jnp.float32),
            pltpu.VMEM((CHUNK,), jnp.float32),
        ],
        compiler_params=pltpu.CompilerParams(needs_layout_passes=False),
    )
    return f(inv, idx, ex4)


@jax.jit
def kernel(q, k, attn, index):
    qT = q.reshape(N_EDGES, DQK).T
    kT = k.reshape(N_EDGES, DQK).T
    a = attn.reshape(N_HEAD, 2 * D_HEAD)
    aq, ak = a[:, :D_HEAD], a[:, D_HEAD:]
    wqc = aq.reshape(DQK, 1)
    wkc = ak.reshape(DQK, 1)
    ll = jnp.arange(DQK)
    hh = jnp.arange(N_HEAD)
    psum = ((ll[None, :] // D_HEAD) == hh[:, None]).astype(jnp.bfloat16)
    idx = index.astype(jnp.int32)

    ex4 = _stage1(qT, kT, wqc, wkc, psum)
    partials = _stage2(ex4, idx)
    inv = _stage3(partials)
    outp = _stage4(inv, idx, ex4)
    return jnp.transpose(outp.reshape(1, N_HEAD, N_EDGES), (0, 2, 1))


# speedup vs baseline: 1.0892x; 1.0892x over previous
import jax
import jax.numpy as jnp
from jax import lax
from jax.experimental import pallas as pl
from jax.experimental.pallas import tpu as pltpu
from jax.experimental.pallas import tpu_sc as plsc

N_NODES = 50000
N_EDGES = 1600000
N_HEAD = 4
D_HEAD = 8
DQK = N_HEAD * D_HEAD
ROWS = N_EDGES // 4
OROWS = N_EDGES * N_HEAD // 128

N_SLOTS = 8
CROWS = 1600
CHUNK = CROWS * 4
N_CHUNKS = N_EDGES // CHUNK
MAX_J = (N_CHUNKS + N_SLOTS - 1) // N_SLOTS
NPAD = 51200
LANES = 16


def _score_body(q_ref, k_ref, wq_ref, wk_ref, p_ref, o_ref):
    u = q_ref[...] * wq_ref[...] + k_ref[...] * wk_ref[...]
    hi = u.astype(jnp.bfloat16)
    lo = (u - hi.astype(jnp.float32)).astype(jnp.bfloat16)
    p = p_ref[...]
    s = lax.dot_general(p, hi, (((1,), (0,)), ((), ())),
                        preferred_element_type=jnp.float32)
    s = s + lax.dot_general(p, lo, (((1,), (0,)), ((), ())),
                            preferred_element_type=jnp.float32)
    o_ref[...] = jnp.exp(jnp.maximum(s, 0.0))


def _stage1(qT, kT, wqc, wkc, psum, be=32000):
    grid = N_EDGES // be
    return pl.pallas_call(
        _score_body,
        grid=(grid,),
        in_specs=[
            pl.BlockSpec((DQK, be), lambda i: (0, i)),
            pl.BlockSpec((DQK, be), lambda i: (0, i)),
            pl.BlockSpec((DQK, 1), lambda i: (0, 0)),
            pl.BlockSpec((DQK, 1), lambda i: (0, 0)),
            pl.BlockSpec((N_HEAD, DQK), lambda i: (0, 0)),
        ],
        out_specs=pl.BlockSpec((N_HEAD, be), lambda i: (0, i)),
        out_shape=jax.ShapeDtypeStruct((N_HEAD, N_EDGES), jnp.float32),
    )(qT, kT, wqc, wkc, psum)


def _scatter_body(ex4_hbm, idx_hbm, part_hbm, table, idx_buf, val4_buf):
    wid = lax.axis_index("s") * 2 + lax.axis_index("c")
    head = wid // N_SLOTS
    slot = wid % N_SLOTS

    def zero_step(i, _):
        table[i >> 3, pl.ds((i & 7) * LANES, LANES)] = jnp.zeros((LANES,), jnp.float32)
        return 0
    lax.fori_loop(0, NPAD // LANES, zero_step, 0)

    def chunk_step(j, _):
        c = slot + j * N_SLOTS

        @pl.when(c < N_CHUNKS)
        def _():
            pltpu.sync_copy(idx_hbm.at[pl.ds(c * CHUNK, CHUNK)], idx_buf)
            pltpu.sync_copy(ex4_hbm.at[:, pl.ds(c * CHUNK, CHUNK)], val4_buf)

            def scat_step(t, _):
                iv = idx_buf[pl.ds(t * LANES, LANES)]
                xv = val4_buf[head, pl.ds(t * LANES, LANES)]
                plsc.addupdate_scatter(table, [iv >> 7, iv & 127], xv)
                return 0
            lax.fori_loop(0, CHUNK // LANES, scat_step, 0)
        return 0
    lax.fori_loop(0, MAX_J, chunk_step, 0)

    pltpu.sync_copy(table, part_hbm.at[pl.ds(wid * (NPAD // 128), NPAD // 128), :])


def _stage2(ex4, idx):
    mesh = plsc.VectorSubcoreMesh(core_axis_name="c", subcore_axis_name="s")
    f = pl.kernel(
        _scatter_body,
        out_type=jax.ShapeDtypeStruct((32 * NPAD // 128, 128), jnp.float32),
        mesh=mesh,
        scratch_types=[
            pltpu.VMEM((NPAD // 128, 128), jnp.float32),
            pltpu.VMEM((CHUNK,), jnp.int32),
            pltpu.VMEM((N_HEAD, CHUNK), jnp.float32),
        ],
        compiler_params=pltpu.CompilerParams(needs_layout_passes=False),
    )
    return f(ex4, idx)


def _inv_body(p_ref, o_ref):
    p = p_ref[...].reshape(N_HEAD, N_SLOTS, NPAD // 128, 128)
    d = jnp.sum(p, axis=1)
    o_ref[...] = (1.0 / (d + 1e-16)).reshape(N_HEAD * NPAD // 128, 128)


def _stage3(partials):
    return pl.pallas_call(
        _inv_body,
        out_shape=jax.ShapeDtypeStruct((N_HEAD * NPAD // 128, 128), jnp.float32),
    )(partials)


def _gather_body(inv_hbm, idx_hbm, ex4_hbm, out_hbm, table, idx_buf, val4_buf, g_buf):
    wid = lax.axis_index("s") * 2 + lax.axis_index("c")
    head = wid // N_SLOTS
    slot = wid % N_SLOTS

    pltpu.sync_copy(inv_hbm.at[pl.ds(head * (NPAD // 128), NPAD // 128), :], table)
    def chunk_step(j, _):
        c = slot + j * N_SLOTS

        @pl.when(c < N_CHUNKS)
        def _():
            pltpu.sync_copy(idx_hbm.at[pl.ds(c * CHUNK, CHUNK)], idx_buf)
            pltpu.sync_copy(ex4_hbm.at[:, pl.ds(c * CHUNK, CHUNK)], val4_buf)

            def gat_step(t, _):
                iv = idx_buf[pl.ds(t * LANES, LANES)]
                gv = plsc.load_gather(table, [iv >> 7, iv & 127])
                xv = val4_buf[head, pl.ds(t * LANES, LANES)]
                g_buf[pl.ds(t * LANES, LANES)] = gv * xv
                return 0
            lax.fori_loop(0, CHUNK // LANES, gat_step, 0)
            pltpu.sync_copy(g_buf, out_hbm.at[pl.ds(head * N_EDGES + c * CHUNK, CHUNK)])
        return 0
    lax.fori_loop(0, MAX_J, chunk_step, 0)


def _stage4(inv, idx, ex4):
    mesh = plsc.VectorSubcoreMesh(core_axis_name="c", subcore_axis_name="s")
    f = pl.kernel(
        _gather_body,
        out_type=jax.ShapeDtypeStruct((N_HEAD * N_EDGES,), jnp.float32),
        mesh=mesh,
        scratch_types=[
            pltpu.VMEM((NPAD // 128, 128), jnp.float32),
            pltpu.VMEM((CHUNK,), jnp.int32),
            pltpu.VMEM((N_HEAD, CHUNK), jnp.float32),
            pltpu.VMEM((CHUNK,), jnp.float32),
        ],
        compiler_params=pltpu.CompilerParams(needs_layout_passes=False),
    )
    return f(inv, idx, ex4)


@jax.jit
def kernel(q, k, attn, index):
    qT = q.reshape(N_EDGES, DQK).T
    kT = k.reshape(N_EDGES, DQK).T
    a = attn.reshape(N_HEAD, 2 * D_HEAD)
    aq, ak = a[:, :D_HEAD], a[:, D_HEAD:]
    wqc = aq.reshape(DQK, 1)
    wkc = ak.reshape(DQK, 1)
    ll = jnp.arange(DQK)
    hh = jnp.arange(N_HEAD)
    psum = ((ll[None, :] // D_HEAD) == hh[:, None]).astype(jnp.bfloat16)
    idx = index.astype(jnp.int32)

    ex4 = _stage1(qT, kT, wqc, wkc, psum)
    partials = _stage2(ex4, idx)
    inv = _stage3(partials)
    outp = _stage4(inv, idx, ex4)
    return jnp.transpose(outp.reshape(1, N_HEAD, N_EDGES), (0, 2, 1))
